# baseline (speedup 1.0000x reference)
import functools

import jax
import jax.numpy as jnp
from jax import lax
from jax.experimental import pallas as pl
from jax.experimental.pallas import tpu as pltpu
from jax.experimental.pallas import tpu_sc as plsc

B = 16384
NUM_CLASSES = 1024
IN_DIM = 3
OUT_DIM = 768
LANES = 16
NUM_CORES = 2
NUM_SUBCORES = 16
NW = NUM_CORES * NUM_SUBCORES
BPW = B // NW
K = 32
NCHUNK = BPW // K
NPAIR = NCHUNK // 2
NG = OUT_DIM // (2 * LANES)
ROW = IN_DIM * OUT_DIM


def _body(wt_hbm, vad_hbm, labels_hbm, out_hbm, idx_v, vad_v, rows_a, rows_b,
          out_a, out_b, sem_a, sem_b, sem_oa, sem_ob):
    wid = lax.axis_index("s") * NUM_CORES + lax.axis_index("c")
    base = wid * BPW

    pltpu.sync_copy(labels_hbm.at[pl.ds(base, BPW)], idx_v)
    pltpu.sync_copy(vad_hbm.at[pl.ds(base * IN_DIM, BPW * IN_DIM)], vad_v)

    GT = 4

    def sample_body(rows, out_v, cbase, s):
        p = jnp.full((LANES,), IN_DIM * (cbase + s), jnp.int32)
        vs = [plsc.pack(v, v, format=plsc.PackFormat.INTERLEAVED)
              for v in (plsc.load_gather(vad_v, [p]),
                        plsc.load_gather(vad_v, [p + 1]),
                        plsc.load_gather(vad_v, [p + 2]))]
        for g in range(NG // GT):
            ws = []
            for j in range(GT):
                gg = g * GT + j
                ws.append([
                    plsc.bitcast(
                        rows[s, pl.ds(i * (OUT_DIM // 2) + LANES * gg,
                                      LANES)],
                        jnp.bfloat16)
                    for i in range(IN_DIM)])
            for j in range(GT):
                gg = g * GT + j
                w0, w1, w2 = ws[j]
                acc = w0 * vs[0] + w1 * vs[1] + w2 * vs[2]
                a, b = plsc.unpack(
                    acc, format=plsc.PackFormat.INTERLEAVED,
                    preferred_element_type=jnp.float32)
                out_v[s, pl.ds(2 * LANES * gg, LANES)] = a
                out_v[s, pl.ds(2 * LANES * gg + LANES, LANES)] = b

    def issue_gather(c, rows, sem):
        return pltpu.async_copy(
            wt_hbm.at[idx_v.at[pl.ds(c * K, K)]], rows, sem)

    def out_copy(c, out_v, sem_o):
        return pltpu.make_async_copy(
            out_v, out_hbm.at[pl.ds(base + c * K, K)], sem_o)

    def half(c, c_next, rows, sem, rows_next, sem_next, out_v, sem_o):
        pltpu.make_async_copy(
            wt_hbm.at[idx_v.at[pl.ds(c * K, K)]], rows, sem).wait()
        issue_gather(c_next, rows_next, sem_next)
        cbase = pl.multiple_of(c * K, K)
        plsc.parallel_loop(0, K, step=1, unroll=2)(
            functools.partial(sample_body, rows, out_v, cbase))
        out_copy(c, out_v, sem_o).start()

    issue_gather(0, rows_a, sem_a)

    def pair_body(j, carry):
        c0 = j * 2

        @pl.when(j > 0)
        def _drain_prev():
            out_copy(c0 - 2, out_a, sem_oa).wait()
            out_copy(c0 - 1, out_b, sem_ob).wait()

        half(c0, c0 + 1, rows_a, sem_a, rows_b, sem_b, out_a, sem_oa)
        half(c0 + 1, (c0 + 2) % NCHUNK, rows_b, sem_b, rows_a, sem_a,
             out_b, sem_ob)
        return carry

    lax.fori_loop(0, NPAIR, pair_body, 0)
    pltpu.make_async_copy(
        wt_hbm.at[idx_v.at[pl.ds(0, K)]], rows_a, sem_a).wait()
    out_copy(NCHUNK - 2, out_a, sem_oa).wait()
    out_copy(NCHUNK - 1, out_b, sem_ob).wait()


@jax.jit
def _run(wt, vad, labels):
    mesh = plsc.VectorSubcoreMesh(core_axis_name="c", subcore_axis_name="s")
    kfn = pl.kernel(
        _body,
        out_type=jax.ShapeDtypeStruct((B, OUT_DIM), jnp.float32),
        mesh=mesh,
        compiler_params=pltpu.CompilerParams(needs_layout_passes=False),
        scratch_types=[
            pltpu.VMEM((BPW,), jnp.int32),
            pltpu.VMEM((BPW * IN_DIM,), jnp.float32),
            pltpu.VMEM((K, ROW // 2), jnp.int32),
            pltpu.VMEM((K, ROW // 2), jnp.int32),
            pltpu.VMEM((K, OUT_DIM), jnp.float32),
            pltpu.VMEM((K, OUT_DIM), jnp.float32),
            pltpu.SemaphoreType.DMA,
            pltpu.SemaphoreType.DMA,
            pltpu.SemaphoreType.DMA,
            pltpu.SemaphoreType.DMA,
        ],
    )
    return kfn(wt, vad, labels)


def kernel(vad, labels, W_blocks):
    wtp = W_blocks.transpose(0, 2, 1).reshape(
        NUM_CLASSES, IN_DIM, NG, 2, LANES).swapaxes(3, 4)
    wt = jax.lax.bitcast_convert_type(
        wtp.astype(jnp.bfloat16).reshape(NUM_CLASSES, ROW // 2, 2),
        jnp.int32)
    return _run(wt, vad.reshape(-1), labels.astype(jnp.int32))


# speedup vs baseline: 1.0018x; 1.0018x over previous
import functools

import jax
import jax.numpy as jnp
from jax import lax
from jax.experimental import pallas as pl
from jax.experimental.pallas import tpu as pltpu
from jax.experimental.pallas import tpu_sc as plsc

B = 16384
NUM_CLASSES = 1024
IN_DIM = 3
OUT_DIM = 768
LANES = 16
NUM_CORES = 2
NUM_SUBCORES = 16
NW = NUM_CORES * NUM_SUBCORES
BPW = B // NW
K = 32
NCHUNK = BPW // K
NPAIR = NCHUNK // 2
NG = OUT_DIM // (2 * LANES)
ROW = IN_DIM * OUT_DIM


def _body(wt_hbm, vad_hbm, labels_hbm, out_hbm, idx_v, vad_v, rows_a, rows_b,
          out_a, out_b, sem_a, sem_b, sem_oa, sem_ob):
    wid = lax.axis_index("s") * NUM_CORES + lax.axis_index("c")
    base = wid * BPW

    pltpu.sync_copy(labels_hbm.at[pl.ds(base, BPW)], idx_v)
    pltpu.sync_copy(vad_hbm.at[pl.ds(base * IN_DIM, BPW * IN_DIM)], vad_v)

    GT = 8

    def sample_body(rows, out_v, cbase, s):
        p = jnp.full((LANES,), IN_DIM * (cbase + s), jnp.int32)
        vs = [plsc.pack(v, v, format=plsc.PackFormat.INTERLEAVED)
              for v in (plsc.load_gather(vad_v, [p]),
                        plsc.load_gather(vad_v, [p + 1]),
                        plsc.load_gather(vad_v, [p + 2]))]
        for g in range(NG // GT):
            ws = []
            for j in range(GT):
                gg = g * GT + j
                ws.append([
                    plsc.bitcast(
                        rows[s, pl.ds(i * (OUT_DIM // 2) + LANES * gg,
                                      LANES)],
                        jnp.bfloat16)
                    for i in range(IN_DIM)])
            for j in range(GT):
                gg = g * GT + j
                w0, w1, w2 = ws[j]
                acc = w0 * vs[0] + w1 * vs[1] + w2 * vs[2]
                a, b = plsc.unpack(
                    acc, format=plsc.PackFormat.INTERLEAVED,
                    preferred_element_type=jnp.float32)
                out_v[s, pl.ds(2 * LANES * gg, LANES)] = a
                out_v[s, pl.ds(2 * LANES * gg + LANES, LANES)] = b

    def issue_gather(c, rows, sem):
        return pltpu.async_copy(
            wt_hbm.at[idx_v.at[pl.ds(c * K, K)]], rows, sem)

    def out_copy(c, out_v, sem_o):
        return pltpu.make_async_copy(
            out_v, out_hbm.at[pl.ds(base + c * K, K)], sem_o)

    def half(c, c_next, rows, sem, rows_next, sem_next, out_v, sem_o):
        pltpu.make_async_copy(
            wt_hbm.at[idx_v.at[pl.ds(c * K, K)]], rows, sem).wait()
        issue_gather(c_next, rows_next, sem_next)
        cbase = pl.multiple_of(c * K, K)
        plsc.parallel_loop(0, K, step=1, unroll=2)(
            functools.partial(sample_body, rows, out_v, cbase))
        out_copy(c, out_v, sem_o).start()

    issue_gather(0, rows_a, sem_a)

    def pair_body(j, carry):
        c0 = j * 2

        @pl.when(j > 0)
        def _drain_prev():
            out_copy(c0 - 2, out_a, sem_oa).wait()
            out_copy(c0 - 1, out_b, sem_ob).wait()

        half(c0, c0 + 1, rows_a, sem_a, rows_b, sem_b, out_a, sem_oa)
        half(c0 + 1, (c0 + 2) % NCHUNK, rows_b, sem_b, rows_a, sem_a,
             out_b, sem_ob)
        return carry

    lax.fori_loop(0, NPAIR, pair_body, 0)
    pltpu.make_async_copy(
        wt_hbm.at[idx_v.at[pl.ds(0, K)]], rows_a, sem_a).wait()
    out_copy(NCHUNK - 2, out_a, sem_oa).wait()
    out_copy(NCHUNK - 1, out_b, sem_ob).wait()


@jax.jit
def _run(wt, vad, labels):
    mesh = plsc.VectorSubcoreMesh(core_axis_name="c", subcore_axis_name="s")
    kfn = pl.kernel(
        _body,
        out_type=jax.ShapeDtypeStruct((B, OUT_DIM), jnp.float32),
        mesh=mesh,
        compiler_params=pltpu.CompilerParams(needs_layout_passes=False),
        scratch_types=[
            pltpu.VMEM((BPW,), jnp.int32),
            pltpu.VMEM((BPW * IN_DIM,), jnp.float32),
            pltpu.VMEM((K, ROW // 2), jnp.int32),
            pltpu.VMEM((K, ROW // 2), jnp.int32),
            pltpu.VMEM((K, OUT_DIM), jnp.float32),
            pltpu.VMEM((K, OUT_DIM), jnp.float32),
            pltpu.SemaphoreType.DMA,
            pltpu.SemaphoreType.DMA,
            pltpu.SemaphoreType.DMA,
            pltpu.SemaphoreType.DMA,
        ],
    )
    return kfn(wt, vad, labels)


def kernel(vad, labels, W_blocks):
    wtp = W_blocks.transpose(0, 2, 1).reshape(
        NUM_CLASSES, IN_DIM, NG, 2, LANES).swapaxes(3, 4)
    wt = jax.lax.bitcast_convert_type(
        wtp.astype(jnp.bfloat16).reshape(NUM_CLASSES, ROW // 2, 2),
        jnp.int32)
    return _run(wt, vad.reshape(-1), labels.astype(jnp.int32))
